# baseline (speedup 1.0000x reference)
import jax
import jax.numpy as jnp
import numpy as np
from jax import lax
from jax.experimental import pallas as pl
from jax.experimental.pallas import tpu as pltpu
from jax.experimental.pallas import tpu_sc as plsc

N = 10000
NROWS = 10240
NPAD = 16384
E = 320000
D = 128
G = 64
C = 2
CH = 64
CPT = 160
EPADT = 32 * CPT * CH
EPS = 1e-5

_q = np.arange(D) // 32
_r = np.arange(D) % 32
PHI = (32 * _q + _r // 2 + 16 * (_r % 2)).astype(np.int32)

_MESH = plsc.VectorSubcoreMesh(core_axis_name="c", subcore_axis_name="s")
_F32 = jnp.float32
_BF16 = jnp.bfloat16
_SC_PARAMS = pltpu.CompilerParams(needs_layout_passes=False,
                                  use_tc_tiling_on_sc=False)


def _deg_body(dstr, ewr, deg0, deg1, didx_v, ewv, ibuf, dacc):
    cid = lax.axis_index("c")
    sid = lax.axis_index("s")
    wid = cid * 16 + sid
    val = jnp.where(cid == 0, 1.0, 0.0).astype(_F32)
    ones = jnp.full((16,), 1.0, _F32) * val
    for i in range(64):
        ibuf[pl.ds(i * 16, 16)] = ones
    pltpu.sync_copy(ibuf, dacc.at[pl.ds(sid * 1024, 1024)])
    pltpu.sync_copy(dstr.at[wid], didx_v)
    pltpu.sync_copy(ewr.at[wid], ewv)
    plsc.subcore_barrier()

    def chunk(g, c1):
        pltpu.sync_copy(ewv.at[g], dacc.at[didx_v.at[g]], add=True)
        return c1

    lax.fori_loop(0, CPT, chunk, 0)
    plsc.subcore_barrier()

    @pl.when(cid == 0)
    def _():
        pltpu.sync_copy(dacc.at[pl.ds(sid * 1024, 1024)],
                        deg0.at[pl.ds(sid * 1024, 1024)])

    @pl.when(cid == 1)
    def _():
        pltpu.sync_copy(dacc.at[pl.ds(sid * 1024, 1024)],
                        deg1.at[pl.ds(sid * 1024, 1024)])


_deg_call = pl.kernel(
    _deg_body,
    out_type=[jax.ShapeDtypeStruct((NPAD,), _F32),
              jax.ShapeDtypeStruct((NPAD,), _F32)],
    mesh=_MESH,
    compiler_params=_SC_PARAMS,
    scratch_types=[
        pltpu.VMEM((CPT, CH), jnp.int32),
        pltpu.VMEM((CPT, CH), _F32),
        pltpu.VMEM((1024,), _F32),
        pltpu.VMEM_SHARED((NPAD,), _F32),
    ],
)


def _scale(rows_bf, rowsf, ewst, g):
    for j in range(CH // 16):
        nv = ewst[g, pl.ds(j * 16, 16)]
        for t in range(16):
            e = j * 16 + t
            ne = nv[t]
            for q in range(2):
                ab = plsc.bitcast(rows_bf[e, pl.ds(q * 16, 16)], _BF16)
                a, b = plsc.unpack(ab, format=plsc.PackFormat.INTERLEAVED)
                rowsf[e, pl.ds(q * 32, 16)] = a * ne
                rowsf[e, pl.ds(q * 32 + 16, 16)] = b * ne


def _scat_body(hsbh, srcr, dstr, ewr, part0, part1,
               sidx_v, didx_v, ewst, rows_a, rows_b, rowsf_a, rowsf_b,
               shex, acc, sem_a, sem_b, sem_sa, sem_sb):
    cid = lax.axis_index("c")
    sid = lax.axis_index("s")
    wid = cid * 16 + sid
    @pl.when(sid < 15)
    def _():
        pltpu.sync_copy(hsbh.at[pl.ds(sid * 640, 640)],
                        shex.at[pl.ds(sid * 640, 640)])

    @pl.when(sid == 15)
    def _():
        pltpu.sync_copy(hsbh.at[pl.ds(9600, N - 9600)],
                        shex.at[pl.ds(9600, N - 9600)])

    z16 = jnp.zeros((16,), _F32)

    def zrow(i, carry):
        for q in range(4):
            rowsf_a[i, pl.ds(q * 16, 16)] = z16
        return carry

    lax.fori_loop(0, CH, zrow, 0)
    for k in range(10):
        pltpu.sync_copy(rowsf_a, acc.at[pl.ds(sid * 640 + k * CH, CH)])
    plsc.subcore_barrier()

    HC = CPT // 2
    for half in range(2):
        pltpu.sync_copy(srcr.at[wid, pl.ds(half * HC, HC)], sidx_v)
        pltpu.sync_copy(dstr.at[wid, pl.ds(half * HC, HC)], didx_v)
        pltpu.sync_copy(ewr.at[wid, pl.ds(half * HC, HC)], ewst)
        pltpu.async_copy(shex.at[sidx_v.at[0]], rows_a, sem_a)

        def pair(p, carry):
            g0 = 2 * p
            g1 = g0 + 1
            pltpu.make_async_copy(shex.at[sidx_v.at[g0]], rows_a,
                                  sem_a).wait()
            pltpu.async_copy(shex.at[sidx_v.at[g1]], rows_b, sem_b)
            _scale(rows_a, rowsf_a, ewst, g0)
            ca = pltpu.async_copy(rowsf_a, acc.at[didx_v.at[g0]], sem_sa,
                                  add=True)
            pltpu.make_async_copy(shex.at[sidx_v.at[g1]], rows_b,
                                  sem_b).wait()

            @pl.when(g0 + 2 < HC)
            def _():
                pltpu.async_copy(shex.at[sidx_v.at[g0 + 2]], rows_a, sem_a)

            _scale(rows_b, rowsf_b, ewst, g1)
            cb = pltpu.async_copy(rowsf_b, acc.at[didx_v.at[g1]], sem_sb,
                                  add=True)
            ca.wait()
            cb.wait()
            return carry

        lax.fori_loop(0, HC // 2, pair, 0)
    plsc.subcore_barrier()

    @pl.when(cid == 0)
    def _():
        pltpu.sync_copy(acc.at[pl.ds(sid * 640, 640)],
                        part0.at[pl.ds(sid * 640, 640)])

    @pl.when(cid == 1)
    def _():
        pltpu.sync_copy(acc.at[pl.ds(sid * 640, 640)],
                        part1.at[pl.ds(sid * 640, 640)])


_scat_call = pl.kernel(
    _scat_body,
    out_type=[jax.ShapeDtypeStruct((NROWS, D // 2), _F32),
              jax.ShapeDtypeStruct((NROWS, D // 2), _F32)],
    mesh=_MESH,
    compiler_params=_SC_PARAMS,
    scratch_types=[
        pltpu.VMEM((CPT // 2, CH), jnp.int32),
        pltpu.VMEM((CPT // 2, CH), jnp.int32),
        pltpu.VMEM((CPT // 2, CH), _F32),
        pltpu.VMEM((CH, D // 4), jnp.int32),
        pltpu.VMEM((CH, D // 4), jnp.int32),
        pltpu.VMEM((CH, D // 2), _F32),
        pltpu.VMEM((CH, D // 2), _F32),
        pltpu.VMEM_SHARED((N, D // 4), jnp.int32),
        pltpu.VMEM_SHARED((NROWS, D // 2), _F32),
        pltpu.SemaphoreType.DMA,
        pltpu.SemaphoreType.DMA,
        pltpu.SemaphoreType.DMA,
        pltpu.SemaphoreType.DMA,
    ],
)


def _h0_body(x_ref, w_ref, wp_ref, deg0_ref, deg1_ref,
             hs_ref, hsb_ref, dinv_ref):
    d = deg0_ref[...] + deg1_ref[...]
    dv = jnp.where(d > 0, lax.rsqrt(d), 0.0)
    dinv_ref[...] = dv
    hs_ref[...] = jnp.dot(x_ref[...], w_ref[...],
                          preferred_element_type=_F32) * dv
    hsb_ref[...] = (jnp.dot(x_ref[...], wp_ref[...],
                            preferred_element_type=_F32) * dv).astype(_BF16)


_h0_call = pl.pallas_call(
    _h0_body,
    out_shape=[jax.ShapeDtypeStruct((N, D), _F32),
               jax.ShapeDtypeStruct((N, D), _BF16),
               jax.ShapeDtypeStruct((N, 1), _F32)],
)


def _epi1_body(p0l_ref, p1l_ref, p0r_ref, p1r_ref, hs_ref, dinv_ref,
               b_ref, g_ref, bt_ref, w_ref, wp_ref, hs1_ref, hs1b_ref):
    dv = dinv_ref[...]
    p = jnp.concatenate([p0l_ref[0:N, :] + p1l_ref[0:N, :],
                         p0r_ref[0:N, :] + p1r_ref[0:N, :]], axis=1)
    y = (p + hs_ref[...]) * dv + b_ref[...]
    y = jnp.maximum(y, 0.0)
    m = jnp.mean(y, axis=0, keepdims=True)
    v = jnp.mean(y * y, axis=0, keepdims=True) - m * m
    a = g_ref[...] * lax.rsqrt(v + EPS)
    z = (y - m) * a + bt_ref[...]
    hs1_ref[...] = jnp.dot(z, w_ref[...], preferred_element_type=_F32) * dv
    hs1b_ref[...] = (jnp.dot(z, wp_ref[...],
                             preferred_element_type=_F32) * dv).astype(_BF16)


_epi1_call = pl.pallas_call(
    _epi1_body,
    out_shape=[jax.ShapeDtypeStruct((N, D), _F32),
               jax.ShapeDtypeStruct((N, D), _BF16)],
)


def _epi2_body(p0l_ref, p1l_ref, p0r_ref, p1r_ref, hs_ref, dinv_ref,
               b_ref, g_ref, bt_ref, batch_ref, wl1_ref, bl1_ref,
               wl2_ref, bl2_ref, out_ref):
    dv = dinv_ref[...]
    p = jnp.concatenate([p0l_ref[0:N, :] + p1l_ref[0:N, :],
                         p0r_ref[0:N, :] + p1r_ref[0:N, :]], axis=1)
    y = (p + hs_ref[...]) * dv + b_ref[...]
    y = jnp.maximum(y, 0.0)
    m = jnp.mean(y, axis=0, keepdims=True)
    v = jnp.mean(y * y, axis=0, keepdims=True) - m * m
    a = g_ref[...] * lax.rsqrt(v + EPS)
    z = (y - m) * a + bt_ref[...]
    oh = (batch_ref[...] == lax.broadcasted_iota(jnp.int32, (G, 1), 0))
    pooled = jnp.dot(oh.astype(_F32), z, preferred_element_type=_F32,
                     precision=lax.Precision.HIGHEST)
    h2 = jnp.maximum(
        jnp.dot(pooled, wl1_ref[...], preferred_element_type=_F32)
        + bl1_ref[...], 0.0)
    logits = (jnp.dot(h2, wl2_ref[...], preferred_element_type=_F32)
              + bl2_ref[...])
    mx = jnp.max(logits, axis=-1, keepdims=True)
    sh = logits - mx
    out_ref[...] = sh - jnp.log(jnp.sum(jnp.exp(sh), axis=-1, keepdims=True))


_epi2_call = pl.pallas_call(
    _epi2_body,
    out_shape=jax.ShapeDtypeStruct((G, C), _F32),
)


def kernel(x, edge_index, batch, edge_attr,
           W0, b0, W1, b1, g0, bt0, g1, bt1, Wl1, bl1, Wl2, bl2):
    src = edge_index[0].astype(jnp.int32)
    dst = edge_index[1].astype(jnp.int32)
    ew = edge_attr[:, 0]
    pad = EPADT - E
    pad_idx = (jnp.arange(pad, dtype=jnp.int32) * 37) % N
    src_p = jnp.concatenate([src, pad_idx])
    dst_p = jnp.concatenate([dst, pad_idx])
    ew_p = jnp.concatenate([ew, jnp.zeros((pad,), _F32)])
    srcr = src_p.reshape(32, CPT, CH)
    dstr = dst_p.reshape(32, CPT, CH)
    ewr = ew_p.reshape(32, CPT, CH)

    W0p = W0[:, PHI]
    W1p = W1[:, PHI]

    deg0, deg1 = _deg_call(dstr, ewr)
    hs0, hs0b, dinv_col = _h0_call(x, W0, W0p, deg0[:N].reshape(N, 1),
                                   deg1[:N].reshape(N, 1))

    def _as_i32(hb):
        return lax.bitcast_convert_type(hb.reshape(N, D // 2, 2), jnp.int32)

    hv0 = _as_i32(hs0b)
    p0l, p1l = _scat_call(hv0[:, 0:32], srcr, dstr, ewr)
    p0r, p1r = _scat_call(hv0[:, 32:64], srcr, dstr, ewr)
    hs1, hs1b = _epi1_call(p0l, p1l, p0r, p1r, hs0, dinv_col,
                           b0.reshape(1, D), g0.reshape(1, D),
                           bt0.reshape(1, D), W1, W1p)
    hv1 = _as_i32(hs1b)
    q0l, q1l = _scat_call(hv1[:, 0:32], srcr, dstr, ewr)
    q0r, q1r = _scat_call(hv1[:, 32:64], srcr, dstr, ewr)
    out = _epi2_call(q0l, q1l, q0r, q1r, hs1, dinv_col, b1.reshape(1, D),
                     g1.reshape(1, D), bt1.reshape(1, D),
                     batch.astype(jnp.int32).reshape(1, N),
                     Wl1, bl1.reshape(1, D), Wl2, bl2.reshape(1, C))
    return out


# speedup vs baseline: 1.1232x; 1.1232x over previous
import jax
import jax.numpy as jnp
import numpy as np
from jax import lax
from jax.experimental import pallas as pl
from jax.experimental.pallas import tpu as pltpu
from jax.experimental.pallas import tpu_sc as plsc

N = 10000
NROWS = 10240
NPAD = 16384
E = 320000
D = 128
G = 64
C = 2
CH = 64
CPT = 160
EPADT = 32 * CPT * CH
EPS = 1e-5

_q = np.arange(D) // 32
_r = np.arange(D) % 32
PHI = (32 * _q + _r // 2 + 16 * (_r % 2)).astype(np.int32)

_MESH = plsc.VectorSubcoreMesh(core_axis_name="c", subcore_axis_name="s")
_F32 = jnp.float32
_BF16 = jnp.bfloat16
_SC_PARAMS = pltpu.CompilerParams(needs_layout_passes=False,
                                  use_tc_tiling_on_sc=False)


def _deg_body(dstr, ewr, deg0, deg1, didx_v, ewv, ibuf, dacc, sem_d):
    cid = lax.axis_index("c")
    sid = lax.axis_index("s")
    wid = cid * 16 + sid
    val = jnp.where(cid == 0, 1.0, 0.0).astype(_F32)
    ones = jnp.full((16,), 1.0, _F32) * val
    for i in range(64):
        ibuf[pl.ds(i * 16, 16)] = ones
    pltpu.sync_copy(ibuf, dacc.at[pl.ds(sid * 1024, 1024)])
    pltpu.sync_copy(dstr.at[wid], didx_v)
    pltpu.sync_copy(ewr.at[wid], ewv)
    plsc.subcore_barrier()

    def group(gg, c1):
        base = gg * 8
        for u in range(8):
            pltpu.async_copy(ewv.at[base + u], dacc.at[didx_v.at[base + u]],
                             sem_d, add=True)
        for u in range(8):
            pltpu.make_async_copy(ewv.at[base + u],
                                  dacc.at[didx_v.at[base + u]],
                                  sem_d).wait()
        return c1

    lax.fori_loop(0, CPT // 8, group, 0)
    plsc.subcore_barrier()

    @pl.when(cid == 0)
    def _():
        pltpu.sync_copy(dacc.at[pl.ds(sid * 1024, 1024)],
                        deg0.at[pl.ds(sid * 1024, 1024)])

    @pl.when(cid == 1)
    def _():
        pltpu.sync_copy(dacc.at[pl.ds(sid * 1024, 1024)],
                        deg1.at[pl.ds(sid * 1024, 1024)])


_deg_call = pl.kernel(
    _deg_body,
    out_type=[jax.ShapeDtypeStruct((NPAD,), _F32),
              jax.ShapeDtypeStruct((NPAD,), _F32)],
    mesh=_MESH,
    compiler_params=_SC_PARAMS,
    scratch_types=[
        pltpu.VMEM((CPT, CH), jnp.int32),
        pltpu.VMEM((CPT, CH), _F32),
        pltpu.VMEM((1024,), _F32),
        pltpu.VMEM_SHARED((NPAD,), _F32),
        pltpu.SemaphoreType.DMA,
    ],
)


def _scale(rows_bf, rowsf, ewst, g):
    for j in range(CH // 16):
        nv = ewst[g, pl.ds(j * 16, 16)]
        for t in range(16):
            e = j * 16 + t
            ne = nv[t]
            for q in range(4):
                ab = plsc.bitcast(rows_bf[e, pl.ds(q * 16, 16)], _BF16)
                a, b = plsc.unpack(ab, format=plsc.PackFormat.INTERLEAVED)
                rowsf[e, pl.ds(q * 32, 16)] = a * ne
                rowsf[e, pl.ds(q * 32 + 16, 16)] = b * ne


def _scat_body(hsb, srcr, dstr, ewr, part0, part1,
               sidx_v, didx_v, ewst, rows_a, rows_b, rowsf_a, rowsf_b, acc,
               sem_a, sem_b, sem_sa, sem_sb):
    cid = lax.axis_index("c")
    sid = lax.axis_index("s")
    wid = cid * 16 + sid
    z16 = jnp.zeros((16,), _F32)

    def zrow(i, carry):
        for q in range(8):
            rowsf_a[i, pl.ds(q * 16, 16)] = z16
        return carry

    lax.fori_loop(0, CH, zrow, 0)
    for k in range(10):
        pltpu.sync_copy(rowsf_a, acc.at[pl.ds(sid * 640 + k * CH, CH)])
    plsc.subcore_barrier()

    HC = CPT // 2
    for half in range(2):
        pltpu.sync_copy(srcr.at[wid, pl.ds(half * HC, HC)], sidx_v)
        pltpu.sync_copy(dstr.at[wid, pl.ds(half * HC, HC)], didx_v)
        pltpu.sync_copy(ewr.at[wid, pl.ds(half * HC, HC)], ewst)
        pltpu.async_copy(hsb.at[sidx_v.at[0]], rows_a, sem_a)

        def pair(p, carry):
            g0 = 2 * p
            g1 = g0 + 1
            pltpu.make_async_copy(hsb.at[sidx_v.at[g0]], rows_a,
                                  sem_a).wait()
            pltpu.async_copy(hsb.at[sidx_v.at[g1]], rows_b, sem_b)
            _scale(rows_a, rowsf_a, ewst, g0)
            ca = pltpu.async_copy(rowsf_a, acc.at[didx_v.at[g0]], sem_sa,
                                  add=True)
            pltpu.make_async_copy(hsb.at[sidx_v.at[g1]], rows_b,
                                  sem_b).wait()

            @pl.when(g0 + 2 < HC)
            def _():
                pltpu.async_copy(hsb.at[sidx_v.at[g0 + 2]], rows_a, sem_a)

            _scale(rows_b, rowsf_b, ewst, g1)
            cb = pltpu.async_copy(rowsf_b, acc.at[didx_v.at[g1]], sem_sb,
                                  add=True)
            ca.wait()
            cb.wait()
            return carry

        lax.fori_loop(0, HC // 2, pair, 0)
    plsc.subcore_barrier()

    @pl.when(cid == 0)
    def _():
        pltpu.sync_copy(acc.at[pl.ds(sid * 640, 640)],
                        part0.at[pl.ds(sid * 640, 640)])

    @pl.when(cid == 1)
    def _():
        pltpu.sync_copy(acc.at[pl.ds(sid * 640, 640)],
                        part1.at[pl.ds(sid * 640, 640)])


_scat_call = pl.kernel(
    _scat_body,
    out_type=[jax.ShapeDtypeStruct((NROWS, D), _F32),
              jax.ShapeDtypeStruct((NROWS, D), _F32)],
    mesh=_MESH,
    compiler_params=_SC_PARAMS,
    scratch_types=[
        pltpu.VMEM((CPT // 2, CH), jnp.int32),
        pltpu.VMEM((CPT // 2, CH), jnp.int32),
        pltpu.VMEM((CPT // 2, CH), _F32),
        pltpu.VMEM((CH, D // 2), jnp.int32),
        pltpu.VMEM((CH, D // 2), jnp.int32),
        pltpu.VMEM((CH, D), _F32),
        pltpu.VMEM((CH, D), _F32),
        pltpu.VMEM_SHARED((NROWS, D), _F32),
        pltpu.SemaphoreType.DMA,
        pltpu.SemaphoreType.DMA,
        pltpu.SemaphoreType.DMA,
        pltpu.SemaphoreType.DMA,
    ],
)


def _h0_body(x_ref, w_ref, wp_ref, deg0_ref, deg1_ref,
             hs_ref, hsb_ref, dinv_ref):
    d = deg0_ref[...] + deg1_ref[...]
    dv = jnp.where(d > 0, lax.rsqrt(d), 0.0)
    dinv_ref[...] = dv
    hs_ref[...] = jnp.dot(x_ref[...], w_ref[...],
                          preferred_element_type=_F32) * dv
    hsb_ref[...] = (jnp.dot(x_ref[...], wp_ref[...],
                            preferred_element_type=_F32) * dv).astype(_BF16)


_h0_call = pl.pallas_call(
    _h0_body,
    out_shape=[jax.ShapeDtypeStruct((N, D), _F32),
               jax.ShapeDtypeStruct((N, D), _BF16),
               jax.ShapeDtypeStruct((N, 1), _F32)],
)


def _epi1_body(p0_ref, p1_ref, hs_ref, dinv_ref, b_ref, g_ref, bt_ref,
               w_ref, wp_ref, hs1_ref, hs1b_ref):
    dv = dinv_ref[...]
    y = (p0_ref[0:N, :] + p1_ref[0:N, :] + hs_ref[...]) * dv + b_ref[...]
    y = jnp.maximum(y, 0.0)
    m = jnp.mean(y, axis=0, keepdims=True)
    v = jnp.mean(y * y, axis=0, keepdims=True) - m * m
    a = g_ref[...] * lax.rsqrt(v + EPS)
    z = (y - m) * a + bt_ref[...]
    hs1_ref[...] = jnp.dot(z, w_ref[...], preferred_element_type=_F32) * dv
    hs1b_ref[...] = (jnp.dot(z, wp_ref[...],
                             preferred_element_type=_F32) * dv).astype(_BF16)


_epi1_call = pl.pallas_call(
    _epi1_body,
    out_shape=[jax.ShapeDtypeStruct((N, D), _F32),
               jax.ShapeDtypeStruct((N, D), _BF16)],
)


def _epi2_body(p0_ref, p1_ref, hs_ref, dinv_ref, b_ref, g_ref, bt_ref,
               batch_ref, wl1_ref, bl1_ref, wl2_ref, bl2_ref, out_ref):
    dv = dinv_ref[...]
    y = (p0_ref[0:N, :] + p1_ref[0:N, :] + hs_ref[...]) * dv + b_ref[...]
    y = jnp.maximum(y, 0.0)
    m = jnp.mean(y, axis=0, keepdims=True)
    v = jnp.mean(y * y, axis=0, keepdims=True) - m * m
    a = g_ref[...] * lax.rsqrt(v + EPS)
    z = (y - m) * a + bt_ref[...]
    oh = (batch_ref[...] == lax.broadcasted_iota(jnp.int32, (G, 1), 0))
    pooled = jnp.dot(oh.astype(_F32), z, preferred_element_type=_F32,
                     precision=lax.Precision.HIGHEST)
    h2 = jnp.maximum(
        jnp.dot(pooled, wl1_ref[...], preferred_element_type=_F32)
        + bl1_ref[...], 0.0)
    logits = (jnp.dot(h2, wl2_ref[...], preferred_element_type=_F32)
              + bl2_ref[...])
    mx = jnp.max(logits, axis=-1, keepdims=True)
    sh = logits - mx
    out_ref[...] = sh - jnp.log(jnp.sum(jnp.exp(sh), axis=-1, keepdims=True))


_epi2_call = pl.pallas_call(
    _epi2_body,
    out_shape=jax.ShapeDtypeStruct((G, C), _F32),
)


def kernel(x, edge_index, batch, edge_attr,
           W0, b0, W1, b1, g0, bt0, g1, bt1, Wl1, bl1, Wl2, bl2):
    src = edge_index[0].astype(jnp.int32)
    dst = edge_index[1].astype(jnp.int32)
    ew = edge_attr[:, 0]
    pad = EPADT - E
    pad_idx = (jnp.arange(pad, dtype=jnp.int32) * 37) % N
    src_p = jnp.concatenate([src, pad_idx])
    dst_p = jnp.concatenate([dst, pad_idx])
    ew_p = jnp.concatenate([ew, jnp.zeros((pad,), _F32)])
    srcr = src_p.reshape(32, CPT, CH)
    dstr = dst_p.reshape(32, CPT, CH)
    ewr = ew_p.reshape(32, CPT, CH)

    W0p = W0[:, PHI]
    W1p = W1[:, PHI]

    deg0, deg1 = _deg_call(dstr, ewr)
    hs0, hs0b, dinv_col = _h0_call(x, W0, W0p, deg0[:N].reshape(N, 1),
                                   deg1[:N].reshape(N, 1))

    def _as_i32(hb):
        return lax.bitcast_convert_type(hb.reshape(N, D // 2, 2), jnp.int32)

    p0, p1 = _scat_call(_as_i32(hs0b), srcr, dstr, ewr)
    hs1, hs1b = _epi1_call(p0, p1, hs0, dinv_col, b0.reshape(1, D),
                           g0.reshape(1, D), bt0.reshape(1, D), W1, W1p)
    q0, q1 = _scat_call(_as_i32(hs1b), srcr, dstr, ewr)
    out = _epi2_call(q0, q1, hs1, dinv_col, b1.reshape(1, D),
                     g1.reshape(1, D), bt1.reshape(1, D),
                     batch.astype(jnp.int32).reshape(1, N),
                     Wl1, bl1.reshape(1, D), Wl2, bl2.reshape(1, C))
    return out
